# Initial kernel scaffold; baseline (speedup 1.0000x reference)
import jax
import jax.numpy as jnp
from jax.experimental import pallas as pl


def kernel(nf, ef, msg_params, red_params, edge_index):
    raise NotImplementedError("write your pallas kernel here")


# speedup vs baseline: 1.8576x; 1.8576x over previous
import functools

import jax
import jax.numpy as jnp
from jax import lax
from jax.experimental import pallas as pl
from jax.experimental.pallas import tpu as pltpu
from jax.experimental.pallas import tpu_sc as plsc

N = 10000
E = 320000
IN_NF = 128
IN_EF = 16
H1 = 32
H2 = 32
OUT_NF = 128

NC, NS, LANES = 2, 16, 16
NW = NC * NS
EPW = E // NW
GC = 128
NGC = EPW // GC
GTAIL = EPW - NGC * GC

SC_CHUNK = 2000
NSC = E // SC_CHUNK
NGRP = SC_CHUNK // LANES

BE = 1280
BN = 1000

def _mesh():
    return plsc.VectorSubcoreMesh(
        core_axis_name="c", subcore_axis_name="s",
        num_cores=NC, num_subcores=NS)


def _leaky(x):
    return jnp.where(x >= 0, x, 0.2 * x)


def _pq_body(nf_ref, ws_ref, wd_ref, p_ref, q_ref):
    nf = nf_ref[...]
    p_ref[...] = jnp.dot(nf, ws_ref[...], preferred_element_type=jnp.float32)
    q_ref[...] = jnp.dot(nf, wd_ref[...], preferred_element_type=jnp.float32)


def _pq_call(nf, w1s, w1d):
    return pl.pallas_call(
        _pq_body,
        out_shape=[jax.ShapeDtypeStruct((N, 64), jnp.float32)] * 2,
    )(nf, w1s, w1d)


@functools.cache
def _gather_kernel_fn():
    return pl.kernel(
        _gather_body,
        out_type=[jax.ShapeDtypeStruct((E, 64), jnp.float32)] * 2,
        mesh=_mesh(),
        scratch_types=[
            pltpu.VMEM((GC,), jnp.int32),
            pltpu.VMEM((GC,), jnp.int32),
            pltpu.VMEM((GC, 64), jnp.float32),
            pltpu.VMEM((GC, 64), jnp.float32),
            pltpu.SemaphoreType.DMA,
            pltpu.SemaphoreType.DMA,
        ],
        compiler_params=pltpu.CompilerParams(use_tc_tiling_on_sc=False),
    )


def _gather_body(p_hbm, q_hbm, src_hbm, dst_hbm, ps_hbm, qd_hbm,
                 sidx, didx, prow, qrow, sem1, sem2):
    wid = lax.axis_index("s") * NC + lax.axis_index("c")
    base0 = wid * EPW

    def do_chunk(base, n):
        pltpu.sync_copy(src_hbm.at[pl.ds(base, n)], sidx.at[pl.ds(0, n)])
        pltpu.sync_copy(dst_hbm.at[pl.ds(base, n)], didx.at[pl.ds(0, n)])
        cp1 = pltpu.async_copy(p_hbm.at[sidx.at[pl.ds(0, n)]],
                               prow.at[pl.ds(0, n)], sem1)
        cp2 = pltpu.async_copy(q_hbm.at[didx.at[pl.ds(0, n)]],
                               qrow.at[pl.ds(0, n)], sem2)
        cp1.wait()
        cp2.wait()
        pltpu.sync_copy(prow.at[pl.ds(0, n)], ps_hbm.at[pl.ds(base, n)])
        pltpu.sync_copy(qrow.at[pl.ds(0, n)], qd_hbm.at[pl.ds(base, n)])

    def chunk_loop(ci, carry):
        do_chunk(base0 + ci * GC, GC)
        return carry

    lax.fori_loop(0, NGC, chunk_loop, 0)
    do_chunk(base0 + NGC * GC, GTAIL)


def _edge_mlp_body(ps_ref, qd_ref, ef_ref,
                   w1e_ref, b1_ref, w2_ref, b2_ref, w3_ref, b3_ref,
                   w4k_ref, b4k_ref, w4f_ref, b4f_ref,
                   out_ref):
    h1 = ps_ref[...] + qd_ref[...]
    h1 += jnp.dot(ef_ref[...], w1e_ref[...], preferred_element_type=jnp.float32)
    h1 = _leaky(h1 + b1_ref[...])
    h2 = _leaky(jnp.dot(h1, w2_ref[...], preferred_element_type=jnp.float32)
                + b2_ref[...])
    h3 = _leaky(jnp.dot(h2, w3_ref[...], preferred_element_type=jnp.float32)
                + b3_ref[...])
    kT = lax.dot_general(w4k_ref[...], h3, (((0,), (1,)), ((), ())),
                         preferred_element_type=jnp.float32)
    kT = 1.0 / (1.0 + jnp.exp(-(kT + b4k_ref[...])))
    fT = lax.dot_general(w4f_ref[...], h3, (((0,), (1,)), ((), ())),
                         preferred_element_type=jnp.float32)
    out_ref[...] = (fT + b4f_ref[...]) * kT


def _edge_mlp_call(ps, qd, ef, w1e, b1, w2, b2, w3, b3, w4k, b4k, w4f, b4f):
    grid = E // BE
    wspec = lambda shape: pl.BlockSpec(shape, lambda i: (0,) * len(shape))
    return pl.pallas_call(
        _edge_mlp_body,
        grid=(grid,),
        in_specs=[
            pl.BlockSpec((BE, 64), lambda i: (i, 0)),
            pl.BlockSpec((BE, 64), lambda i: (i, 0)),
            pl.BlockSpec((BE, IN_EF), lambda i: (i, 0)),
            wspec((IN_EF, 64)), wspec((1, 64)),
            wspec((64, 128)), wspec((1, 128)),
            wspec((128, 64)), wspec((1, 64)),
            wspec((64, 1)), wspec((1, 1)),
            wspec((64, 64)), wspec((64, 1)),
        ],
        out_specs=pl.BlockSpec((64, BE), lambda i: (0, i)),
        out_shape=jax.ShapeDtypeStruct((64, E), jnp.float32),
    )(ps, qd, ef, w1e, b1, w2, b2, w3, b3, w4k, b4k, w4f, b4f)


@functools.cache
def _scatter_kernel_fn():
    return pl.kernel(
        _scatter_body,
        out_type=jax.ShapeDtypeStruct((64 * N,), jnp.float32),
        mesh=_mesh(),
        scratch_types=[
            pltpu.VMEM((N,), jnp.float32),
            pltpu.VMEM((N,), jnp.float32),
            pltpu.VMEM((SC_CHUNK,), jnp.int32),
            pltpu.VMEM((SC_CHUNK,), jnp.float32),
            pltpu.VMEM((SC_CHUNK,), jnp.float32),
        ],
        compiler_params=pltpu.CompilerParams(needs_layout_passes=False),
    )


def _scatter_body(eft_hbm, dst_hbm, agg_hbm, sacc, macc, didx, v1, v2):
    wid = lax.axis_index("s") * NC + lax.axis_index("c")

    def init_loop(i, carry):
        sacc[pl.ds(i * LANES, LANES)] = jnp.zeros((LANES,), jnp.float32)
        macc[pl.ds(i * LANES, LANES)] = jnp.full((LANES,), -jnp.inf,
                                                 jnp.float32)
        return carry

    lax.fori_loop(0, N // LANES, init_loop, 0)

    sum_off = wid * E
    max_off = (32 + wid) * E

    def chunk_loop(ci, carry):
        base = ci * SC_CHUNK
        pltpu.sync_copy(dst_hbm.at[pl.ds(base, SC_CHUNK)], didx)
        pltpu.sync_copy(eft_hbm.at[pl.ds(sum_off + base, SC_CHUNK)], v1)
        pltpu.sync_copy(eft_hbm.at[pl.ds(max_off + base, SC_CHUNK)], v2)

        def grp_loop(gi, c2):
            d = didx[pl.ds(gi * LANES, LANES)]
            s = v1[pl.ds(gi * LANES, LANES)]
            m = v2[pl.ds(gi * LANES, LANES)]
            plsc.addupdate_scatter(sacc, [d], s)
            cur = plsc.load_gather(macc, [d])
            new = jnp.maximum(cur, m)
            plsc.store_scatter(macc, [d], new)
            chk = plsc.load_gather(macc, [d])

            def retry_cond(need):
                return jnp.any(need)

            def retry_body(need):
                cur2 = plsc.load_gather(macc, [d])
                plsc.store_scatter(macc, [d], jnp.maximum(cur2, new),
                                  mask=need)
                chk2 = plsc.load_gather(macc, [d])
                return chk2 < new

            lax.while_loop(retry_cond, retry_body, chk < new)
            return c2

        lax.fori_loop(0, NGRP, grp_loop, 0)
        return carry

    lax.fori_loop(0, NSC, chunk_loop, 0)

    pltpu.sync_copy(sacc, agg_hbm.at[pl.ds(wid * N, N)])
    pltpu.sync_copy(macc, agg_hbm.at[pl.ds((32 + wid) * N, N)])


def _out_mlp_body(nf_ref, sumt_ref, maxt_ref,
                  wa_ref, wb_ref, wc_ref, b1_ref,
                  w2_ref, b2_ref, w3_ref, b3_ref, w4_ref, b4_ref,
                  out_ref):
    h1 = jnp.dot(nf_ref[...], wa_ref[...], preferred_element_type=jnp.float32)
    h1 += lax.dot_general(sumt_ref[...], wb_ref[...], (((0,), (0,)), ((), ())),
                          preferred_element_type=jnp.float32)
    mx = maxt_ref[...]
    mx = jnp.where(jnp.isinf(mx) & (mx < 0), 0.0, mx)
    h1 += lax.dot_general(mx, wc_ref[...], (((0,), (0,)), ((), ())),
                          preferred_element_type=jnp.float32)
    h1 = _leaky(h1 + b1_ref[...])
    h2 = _leaky(jnp.dot(h1, w2_ref[...], preferred_element_type=jnp.float32)
                + b2_ref[...])
    h3 = _leaky(jnp.dot(h2, w3_ref[...], preferred_element_type=jnp.float32)
                + b3_ref[...])
    out_ref[...] = jnp.dot(h3, w4_ref[...],
                           preferred_element_type=jnp.float32) + b4_ref[...]


def _out_mlp_call(nf, sumt, maxt, wa, wb, wc, b1, w2, b2, w3, b3, w4, b4):
    return pl.pallas_call(
        _out_mlp_body,
        out_shape=jax.ShapeDtypeStruct((N, OUT_NF), jnp.float32),
    )(nf, sumt, maxt, wa, wb, wc, b1, w2, b2, w3, b3, w4, b4)


@jax.jit
def kernel(nf, ef, msg_params, red_params, edge_index):
    w1, b1, w2, b2, w3, b3, w4, b4 = msg_params
    wr1, br1, wr2, br2, wr3, br3, wr4, br4 = red_params

    src = edge_index[0].astype(jnp.int32)
    dst = edge_index[1].astype(jnp.int32)

    w1s, w1d, w1e = w1[:IN_NF], w1[IN_NF:2 * IN_NF], w1[2 * IN_NF:]
    w4k, w4f = w4[:, :1], w4[:, 1:]
    b4k, b4f = b4[:1].reshape(1, 1), b4[1:].reshape(64, 1)

    p, q = _pq_call(nf, w1s, w1d)
    ps, qd = _gather_kernel_fn()(p, q, src, dst)
    eft = _edge_mlp_call(ps, qd, ef,
                         w1e, b1.reshape(1, 64), w2, b2.reshape(1, 128),
                         w3, b3.reshape(1, 64), w4k, b4k, w4f, b4f)
    agg = _scatter_kernel_fn()(eft.reshape(64 * E), dst)
    agg = agg.reshape(64, N)
    sumt, maxt = agg[:32], agg[32:]

    wa, wb, wc = wr1[:IN_NF], wr1[IN_NF:IN_NF + 32], wr1[IN_NF + 32:]
    return _out_mlp_call(nf, sumt, maxt,
                         wa, wb, wc, br1.reshape(1, 64),
                         wr2, br2.reshape(1, 128), wr3, br3.reshape(1, 64),
                         wr4, br4.reshape(1, OUT_NF))


# speedup vs baseline: 2.7200x; 1.4643x over previous
import functools

import jax
import jax.numpy as jnp
from jax import lax
from jax.experimental import pallas as pl
from jax.experimental.pallas import tpu as pltpu
from jax.experimental.pallas import tpu_sc as plsc

N = 10000
E = 320000
IN_NF = 128
IN_EF = 16
H1 = 32
H2 = 32
OUT_NF = 128

NC, NS, LANES = 2, 16, 16
NW = NC * NS
EPW = E // NW
GC = 128
NGC = EPW // GC
GTAIL = EPW - NGC * GC

SC_CHUNK = 2000
NSC = E // SC_CHUNK
NGRP = SC_CHUNK // LANES

BE = 1280
BN = 1000

def _mesh():
    return plsc.VectorSubcoreMesh(
        core_axis_name="c", subcore_axis_name="s",
        num_cores=NC, num_subcores=NS)


def _leaky(x):
    return jnp.where(x >= 0, x, 0.2 * x)


def _pq_body(nf_ref, ws_ref, wd_ref, p_ref, q_ref):
    nf = nf_ref[...]
    p_ref[...] = jnp.dot(nf, ws_ref[...], preferred_element_type=jnp.float32)
    q_ref[...] = jnp.dot(nf, wd_ref[...], preferred_element_type=jnp.float32)


def _pq_call(nf, w1s, w1d):
    return pl.pallas_call(
        _pq_body,
        out_shape=[jax.ShapeDtypeStruct((N, 64), jnp.float32)] * 2,
    )(nf, w1s, w1d)


@functools.cache
def _gather_kernel_fn():
    return pl.kernel(
        _gather_body,
        out_type=[jax.ShapeDtypeStruct((E, 64), jnp.float32)] * 2,
        mesh=_mesh(),
        scratch_types=[
            pltpu.VMEM((GC,), jnp.int32),
            pltpu.VMEM((GC,), jnp.int32),
            pltpu.VMEM((GC, 64), jnp.float32),
            pltpu.VMEM((GC, 64), jnp.float32),
            pltpu.SemaphoreType.DMA,
            pltpu.SemaphoreType.DMA,
        ],
        compiler_params=pltpu.CompilerParams(use_tc_tiling_on_sc=False),
    )


def _gather_body(p_hbm, q_hbm, src_hbm, dst_hbm, ps_hbm, qd_hbm,
                 sidx, didx, prow, qrow, sem1, sem2):
    wid = lax.axis_index("s") * NC + lax.axis_index("c")
    base0 = wid * EPW

    def do_chunk(base, n):
        pltpu.sync_copy(src_hbm.at[pl.ds(base, n)], sidx.at[pl.ds(0, n)])
        pltpu.sync_copy(dst_hbm.at[pl.ds(base, n)], didx.at[pl.ds(0, n)])
        cp1 = pltpu.async_copy(p_hbm.at[sidx.at[pl.ds(0, n)]],
                               prow.at[pl.ds(0, n)], sem1)
        cp2 = pltpu.async_copy(q_hbm.at[didx.at[pl.ds(0, n)]],
                               qrow.at[pl.ds(0, n)], sem2)
        cp1.wait()
        cp2.wait()
        pltpu.sync_copy(prow.at[pl.ds(0, n)], ps_hbm.at[pl.ds(base, n)])
        pltpu.sync_copy(qrow.at[pl.ds(0, n)], qd_hbm.at[pl.ds(base, n)])

    def chunk_loop(ci, carry):
        do_chunk(base0 + ci * GC, GC)
        return carry

    lax.fori_loop(0, NGC, chunk_loop, 0)
    do_chunk(base0 + NGC * GC, GTAIL)


def _edge_mlp_body(ps_ref, qd_ref, ef_ref,
                   w1e_ref, b1_ref, w2_ref, b2_ref, w3_ref, b3_ref,
                   w4k_ref, b4k_ref, w4f_ref, b4f_ref,
                   out_ref):
    h1 = ps_ref[...] + qd_ref[...]
    h1 += jnp.dot(ef_ref[...], w1e_ref[...], preferred_element_type=jnp.float32)
    h1 = _leaky(h1 + b1_ref[...])
    h2 = _leaky(jnp.dot(h1, w2_ref[...], preferred_element_type=jnp.float32)
                + b2_ref[...])
    h3 = _leaky(jnp.dot(h2, w3_ref[...], preferred_element_type=jnp.float32)
                + b3_ref[...])
    kT = lax.dot_general(w4k_ref[...], h3, (((0,), (1,)), ((), ())),
                         preferred_element_type=jnp.float32)
    kT = 1.0 / (1.0 + jnp.exp(-(kT + b4k_ref[...])))
    fT = lax.dot_general(w4f_ref[...], h3, (((0,), (1,)), ((), ())),
                         preferred_element_type=jnp.float32)
    out_ref[...] = (fT + b4f_ref[...]) * kT


def _edge_mlp_call(ps, qd, ef, w1e, b1, w2, b2, w3, b3, w4k, b4k, w4f, b4f):
    grid = E // BE
    wspec = lambda shape: pl.BlockSpec(shape, lambda i: (0,) * len(shape))
    return pl.pallas_call(
        _edge_mlp_body,
        grid=(grid,),
        in_specs=[
            pl.BlockSpec((BE, 64), lambda i: (i, 0)),
            pl.BlockSpec((BE, 64), lambda i: (i, 0)),
            pl.BlockSpec((BE, IN_EF), lambda i: (i, 0)),
            wspec((IN_EF, 64)), wspec((1, 64)),
            wspec((64, 128)), wspec((1, 128)),
            wspec((128, 64)), wspec((1, 64)),
            wspec((64, 1)), wspec((1, 1)),
            wspec((64, 64)), wspec((64, 1)),
        ],
        out_specs=pl.BlockSpec((64, BE), lambda i: (0, i)),
        out_shape=jax.ShapeDtypeStruct((64, E), jnp.float32),
    )(ps, qd, ef, w1e, b1, w2, b2, w3, b3, w4k, b4k, w4f, b4f)


@functools.cache
def _scatter_kernel_fn():
    return pl.kernel(
        _scatter_body,
        out_type=jax.ShapeDtypeStruct((64 * N,), jnp.float32),
        mesh=_mesh(),
        scratch_types=[
            pltpu.VMEM((N,), jnp.float32),
            pltpu.VMEM((N,), jnp.float32),
            pltpu.VMEM((SC_CHUNK,), jnp.int32),
            pltpu.VMEM((SC_CHUNK,), jnp.float32),
            pltpu.VMEM((SC_CHUNK,), jnp.float32),
            pltpu.VMEM((SC_CHUNK,), jnp.int32),
            pltpu.VMEM((SC_CHUNK,), jnp.float32),
            pltpu.VMEM((SC_CHUNK,), jnp.float32),
            pltpu.SemaphoreType.DMA,
            pltpu.SemaphoreType.DMA,
        ],
        compiler_params=pltpu.CompilerParams(needs_layout_passes=False),
    )


def _scatter_body(eft_hbm, dst_hbm, agg_hbm, sacc, macc,
                  didx_a, v1_a, v2_a, didx_b, v1_b, v2_b, sem_a, sem_b):
    wid = lax.axis_index("s") * NC + lax.axis_index("c")

    def init_loop(i, carry):
        sacc[pl.ds(i * LANES, LANES)] = jnp.zeros((LANES,), jnp.float32)
        macc[pl.ds(i * LANES, LANES)] = jnp.full((LANES,), -jnp.inf,
                                                 jnp.float32)
        return carry

    lax.fori_loop(0, N // LANES, init_loop, 0)

    sum_off = wid * E
    max_off = (32 + wid) * E

    def issue(base, bufs, sem):
        bd, b1, b2 = bufs
        pltpu.async_copy(dst_hbm.at[pl.ds(base, SC_CHUNK)], bd, sem)
        pltpu.async_copy(eft_hbm.at[pl.ds(sum_off + base, SC_CHUNK)], b1, sem)
        pltpu.async_copy(eft_hbm.at[pl.ds(max_off + base, SC_CHUNK)], b2, sem)

    def drain(bufs, sem):
        bd, b1, b2 = bufs
        pltpu.make_async_copy(dst_hbm.at[pl.ds(0, SC_CHUNK)], bd, sem).wait()
        pltpu.make_async_copy(eft_hbm.at[pl.ds(0, SC_CHUNK)], b1, sem).wait()
        pltpu.make_async_copy(eft_hbm.at[pl.ds(0, SC_CHUNK)], b2, sem).wait()

    def compute(bufs):
        bd, b1, b2 = bufs
        def grp(gi, fail):
            sl = pl.ds(gi * LANES, LANES)
            d = bd[sl]
            s = b1[sl]
            m = b2[sl]
            plsc.addupdate_scatter(sacc, [d], s)
            cur = plsc.load_gather(macc, [d])
            new = jnp.maximum(cur, m)
            plsc.store_scatter(macc, [d], new)
            cur2 = plsc.load_gather(macc, [d])
            plsc.store_scatter(macc, [d], jnp.maximum(cur2, new),
                              mask=cur2 < new)
            chk = plsc.load_gather(macc, [d])
            return fail | (chk < new)

        fail = lax.fori_loop(0, NGRP, grp, jnp.zeros((LANES,), jnp.bool_))

        @pl.when(jnp.any(fail))
        def _fixup():
            def grp2(gi, carry):
                sl = pl.ds(gi * LANES, LANES)
                d = bd[sl]
                m = b2[sl]

                def retry_body(need):
                    c2 = plsc.load_gather(macc, [d])
                    plsc.store_scatter(macc, [d], jnp.maximum(c2, m),
                                      mask=need)
                    c3 = plsc.load_gather(macc, [d])
                    return c3 < m

                lax.while_loop(lambda n: jnp.any(n), retry_body,
                               plsc.load_gather(macc, [d]) < m)
                return carry

            lax.fori_loop(0, NGRP, grp2, 0)

    bufs_a = (didx_a, v1_a, v2_a)
    bufs_b = (didx_b, v1_b, v2_b)
    issue(0, bufs_a, sem_a)

    def pair_loop(ci, carry):
        base_a = (2 * ci) * SC_CHUNK
        base_b = base_a + SC_CHUNK
        base_n = jnp.minimum(base_a + 2 * SC_CHUNK, E - SC_CHUNK)
        drain(bufs_a, sem_a)
        issue(base_b, bufs_b, sem_b)
        compute(bufs_a)
        drain(bufs_b, sem_b)
        issue(base_n, bufs_a, sem_a)
        compute(bufs_b)
        return carry

    lax.fori_loop(0, NSC // 2, pair_loop, 0)
    drain(bufs_a, sem_a)

    pltpu.sync_copy(sacc, agg_hbm.at[pl.ds(wid * N, N)])
    pltpu.sync_copy(macc, agg_hbm.at[pl.ds((32 + wid) * N, N)])


def _out_mlp_body(nf_ref, sumt_ref, maxt_ref,
                  wa_ref, wb_ref, wc_ref, b1_ref,
                  w2_ref, b2_ref, w3_ref, b3_ref, w4_ref, b4_ref,
                  out_ref):
    h1 = jnp.dot(nf_ref[...], wa_ref[...], preferred_element_type=jnp.float32)
    h1 += lax.dot_general(sumt_ref[...], wb_ref[...], (((0,), (0,)), ((), ())),
                          preferred_element_type=jnp.float32)
    mx = maxt_ref[...]
    mx = jnp.where(jnp.isinf(mx) & (mx < 0), 0.0, mx)
    h1 += lax.dot_general(mx, wc_ref[...], (((0,), (0,)), ((), ())),
                          preferred_element_type=jnp.float32)
    h1 = _leaky(h1 + b1_ref[...])
    h2 = _leaky(jnp.dot(h1, w2_ref[...], preferred_element_type=jnp.float32)
                + b2_ref[...])
    h3 = _leaky(jnp.dot(h2, w3_ref[...], preferred_element_type=jnp.float32)
                + b3_ref[...])
    out_ref[...] = jnp.dot(h3, w4_ref[...],
                           preferred_element_type=jnp.float32) + b4_ref[...]


def _out_mlp_call(nf, sumt, maxt, wa, wb, wc, b1, w2, b2, w3, b3, w4, b4):
    return pl.pallas_call(
        _out_mlp_body,
        out_shape=jax.ShapeDtypeStruct((N, OUT_NF), jnp.float32),
    )(nf, sumt, maxt, wa, wb, wc, b1, w2, b2, w3, b3, w4, b4)


@jax.jit
def kernel(nf, ef, msg_params, red_params, edge_index):
    w1, b1, w2, b2, w3, b3, w4, b4 = msg_params
    wr1, br1, wr2, br2, wr3, br3, wr4, br4 = red_params

    src = edge_index[0].astype(jnp.int32)
    dst = edge_index[1].astype(jnp.int32)

    w1s, w1d, w1e = w1[:IN_NF], w1[IN_NF:2 * IN_NF], w1[2 * IN_NF:]
    w4k, w4f = w4[:, :1], w4[:, 1:]
    b4k, b4f = b4[:1].reshape(1, 1), b4[1:].reshape(64, 1)

    p, q = _pq_call(nf, w1s, w1d)
    ps, qd = _gather_kernel_fn()(p, q, src, dst)
    eft = _edge_mlp_call(ps, qd, ef,
                         w1e, b1.reshape(1, 64), w2, b2.reshape(1, 128),
                         w3, b3.reshape(1, 64), w4k, b4k, w4f, b4f)
    agg = _scatter_kernel_fn()(eft.reshape(64 * E), dst)
    agg = agg.reshape(64, N)
    sumt, maxt = agg[:32], agg[32:]

    wa, wb, wc = wr1[:IN_NF], wr1[IN_NF:IN_NF + 32], wr1[IN_NF + 32:]
    return _out_mlp_call(nf, sumt, maxt,
                         wa, wb, wc, br1.reshape(1, 64),
                         wr2, br2.reshape(1, 128), wr3, br3.reshape(1, 64),
                         wr4, br4.reshape(1, OUT_NF))


# speedup vs baseline: 3.1855x; 1.1711x over previous
import functools

import jax
import jax.numpy as jnp
from jax import lax
from jax.experimental import pallas as pl
from jax.experimental.pallas import tpu as pltpu
from jax.experimental.pallas import tpu_sc as plsc

N = 10000
E = 320000
IN_NF = 128
IN_EF = 16
H1 = 32
H2 = 32
OUT_NF = 128

NC, NS, LANES = 2, 16, 16
NW = NC * NS

NSLAB = 2
SE = E // NSLAB
EPW = SE // NW
GC = 128
NGC = EPW // GC
GTAIL = EPW - NGC * GC

SC_CHUNK = 2000
NSC = SE // SC_CHUNK
NGRP = SC_CHUNK // LANES

BE = 1280

def _mesh():
    return plsc.VectorSubcoreMesh(
        core_axis_name="c", subcore_axis_name="s",
        num_cores=NC, num_subcores=NS)


def _leaky(x):
    return jnp.where(x >= 0, x, 0.2 * x)


def _pq_body(nf_ref, ws_ref, wd_ref, p_ref, q_ref):
    nf = nf_ref[...]
    p_ref[...] = jnp.dot(nf, ws_ref[...], preferred_element_type=jnp.float32)
    q_ref[...] = jnp.dot(nf, wd_ref[...], preferred_element_type=jnp.float32)


def _pq_call(nf, w1s, w1d):
    return pl.pallas_call(
        _pq_body,
        out_shape=[jax.ShapeDtypeStruct((N, 64), jnp.float32)] * 2,
    )(nf, w1s, w1d)


@functools.cache
def _gather_kernel_fn():
    return pl.kernel(
        _gather_body,
        out_type=[jax.ShapeDtypeStruct((SE, 64), jnp.float32)] * 2,
        mesh=_mesh(),
        scratch_types=[
            pltpu.VMEM((GC,), jnp.int32),
            pltpu.VMEM((GC,), jnp.int32),
            pltpu.VMEM((GC, 64), jnp.float32),
            pltpu.VMEM((GC, 64), jnp.float32),
            pltpu.SemaphoreType.DMA,
            pltpu.SemaphoreType.DMA,
        ],
        compiler_params=pltpu.CompilerParams(use_tc_tiling_on_sc=False),
    )


def _gather_body(p_hbm, q_hbm, src_hbm, dst_hbm, ps_hbm, qd_hbm,
                 sidx, didx, prow, qrow, sem1, sem2):
    wid = lax.axis_index("s") * NC + lax.axis_index("c")
    base0 = wid * EPW

    def do_chunk(base, n):
        pltpu.sync_copy(src_hbm.at[pl.ds(base, n)], sidx.at[pl.ds(0, n)])
        pltpu.sync_copy(dst_hbm.at[pl.ds(base, n)], didx.at[pl.ds(0, n)])
        cp1 = pltpu.async_copy(p_hbm.at[sidx.at[pl.ds(0, n)]],
                               prow.at[pl.ds(0, n)], sem1)
        cp2 = pltpu.async_copy(q_hbm.at[didx.at[pl.ds(0, n)]],
                               qrow.at[pl.ds(0, n)], sem2)
        cp1.wait()
        cp2.wait()
        pltpu.sync_copy(prow.at[pl.ds(0, n)], ps_hbm.at[pl.ds(base, n)])
        pltpu.sync_copy(qrow.at[pl.ds(0, n)], qd_hbm.at[pl.ds(base, n)])

    def chunk_loop(ci, carry):
        do_chunk(base0 + ci * GC, GC)
        return carry

    lax.fori_loop(0, NGC, chunk_loop, 0)
    do_chunk(base0 + NGC * GC, GTAIL)


def _edge_mlp_body(ps_ref, qd_ref, ef_ref,
                   w1e_ref, b1_ref, w2_ref, b2_ref, w3_ref, b3_ref,
                   w4k_ref, b4k_ref, w4f_ref, b4f_ref,
                   out_ref):
    h1 = ps_ref[...] + qd_ref[...]
    h1 += jnp.dot(ef_ref[...], w1e_ref[...], preferred_element_type=jnp.float32)
    h1 = _leaky(h1 + b1_ref[...])
    h2 = _leaky(jnp.dot(h1, w2_ref[...], preferred_element_type=jnp.float32)
                + b2_ref[...])
    h3 = _leaky(jnp.dot(h2, w3_ref[...], preferred_element_type=jnp.float32)
                + b3_ref[...])
    kT = lax.dot_general(w4k_ref[...], h3, (((0,), (1,)), ((), ())),
                         preferred_element_type=jnp.float32)
    kT = 1.0 / (1.0 + jnp.exp(-(kT + b4k_ref[...])))
    fT = lax.dot_general(w4f_ref[...], h3, (((0,), (1,)), ((), ())),
                         preferred_element_type=jnp.float32)
    out_ref[...] = (fT + b4f_ref[...]) * kT


def _edge_mlp_call(ps, qd, ef, w1e, b1, w2, b2, w3, b3, w4k, b4k, w4f, b4f):
    grid = SE // BE
    wspec = lambda shape: pl.BlockSpec(shape, lambda i: (0,) * len(shape))
    return pl.pallas_call(
        _edge_mlp_body,
        grid=(grid,),
        in_specs=[
            pl.BlockSpec((BE, 64), lambda i: (i, 0)),
            pl.BlockSpec((BE, 64), lambda i: (i, 0)),
            pl.BlockSpec((BE, IN_EF), lambda i: (i, 0)),
            wspec((IN_EF, 64)), wspec((1, 64)),
            wspec((64, 128)), wspec((1, 128)),
            wspec((128, 64)), wspec((1, 64)),
            wspec((64, 1)), wspec((1, 1)),
            wspec((64, 64)), wspec((64, 1)),
        ],
        out_specs=pl.BlockSpec((64, BE), lambda i: (0, i)),
        out_shape=jax.ShapeDtypeStruct((64, SE), jnp.float32),
    )(ps, qd, ef, w1e, b1, w2, b2, w3, b3, w4k, b4k, w4f, b4f)


@functools.cache
def _scatter_kernel_fn():
    return pl.kernel(
        _scatter_body,
        out_type=jax.ShapeDtypeStruct((64 * N,), jnp.float32),
        mesh=_mesh(),
        scratch_types=[
            pltpu.VMEM((N,), jnp.float32),
            pltpu.VMEM((N,), jnp.float32),
            pltpu.VMEM((SC_CHUNK,), jnp.int32),
            pltpu.VMEM((SC_CHUNK,), jnp.float32),
            pltpu.VMEM((SC_CHUNK,), jnp.float32),
            pltpu.VMEM((SC_CHUNK,), jnp.int32),
            pltpu.VMEM((SC_CHUNK,), jnp.float32),
            pltpu.VMEM((SC_CHUNK,), jnp.float32),
            pltpu.SemaphoreType.DMA,
            pltpu.SemaphoreType.DMA,
        ],
        compiler_params=pltpu.CompilerParams(needs_layout_passes=False),
    )


def _scatter_body(eft_hbm, dst_hbm, agg_hbm, sacc, macc,
                  didx_a, v1_a, v2_a, didx_b, v1_b, v2_b, sem_a, sem_b):
    wid = lax.axis_index("s") * NC + lax.axis_index("c")

    def init_loop(i, carry):
        sacc[pl.ds(i * LANES, LANES)] = jnp.zeros((LANES,), jnp.float32)
        macc[pl.ds(i * LANES, LANES)] = jnp.full((LANES,), -jnp.inf,
                                                 jnp.float32)
        return carry

    lax.fori_loop(0, N // LANES, init_loop, 0)

    sum_off = wid * SE
    max_off = (32 + wid) * SE

    def issue(base, bufs, sem):
        bd, b1, b2 = bufs
        pltpu.async_copy(dst_hbm.at[pl.ds(base, SC_CHUNK)], bd, sem)
        pltpu.async_copy(eft_hbm.at[pl.ds(sum_off + base, SC_CHUNK)], b1, sem)
        pltpu.async_copy(eft_hbm.at[pl.ds(max_off + base, SC_CHUNK)], b2, sem)

    def drain(bufs, sem):
        bd, b1, b2 = bufs
        pltpu.make_async_copy(dst_hbm.at[pl.ds(0, SC_CHUNK)], bd, sem).wait()
        pltpu.make_async_copy(eft_hbm.at[pl.ds(0, SC_CHUNK)], b1, sem).wait()
        pltpu.make_async_copy(eft_hbm.at[pl.ds(0, SC_CHUNK)], b2, sem).wait()

    def compute(bufs):
        bd, b1, b2 = bufs
        def grp(gi, fail):
            sl = pl.ds(gi * LANES, LANES)
            d = bd[sl]
            s = b1[sl]
            m = b2[sl]
            plsc.addupdate_scatter(sacc, [d], s)
            cur = plsc.load_gather(macc, [d])
            new = jnp.maximum(cur, m)
            plsc.store_scatter(macc, [d], new)
            cur2 = plsc.load_gather(macc, [d])
            plsc.store_scatter(macc, [d], jnp.maximum(cur2, new),
                              mask=cur2 < new)
            chk = plsc.load_gather(macc, [d])
            return fail | (chk < new)

        fail = lax.fori_loop(0, NGRP, grp, jnp.zeros((LANES,), jnp.bool_))

        @pl.when(jnp.any(fail))
        def _fixup():
            def grp2(gi, carry):
                sl = pl.ds(gi * LANES, LANES)
                d = bd[sl]
                m = b2[sl]

                def retry_body(need):
                    c2 = plsc.load_gather(macc, [d])
                    plsc.store_scatter(macc, [d], jnp.maximum(c2, m),
                                      mask=need)
                    c3 = plsc.load_gather(macc, [d])
                    return c3 < m

                lax.while_loop(lambda n: jnp.any(n), retry_body,
                               plsc.load_gather(macc, [d]) < m)
                return carry

            lax.fori_loop(0, NGRP, grp2, 0)

    bufs_a = (didx_a, v1_a, v2_a)
    bufs_b = (didx_b, v1_b, v2_b)
    issue(0, bufs_a, sem_a)

    def pair_loop(ci, carry):
        base_a = (2 * ci) * SC_CHUNK
        base_b = base_a + SC_CHUNK
        base_n = jnp.minimum(base_a + 2 * SC_CHUNK, SE - SC_CHUNK)
        drain(bufs_a, sem_a)
        issue(base_b, bufs_b, sem_b)
        compute(bufs_a)
        drain(bufs_b, sem_b)
        issue(base_n, bufs_a, sem_a)
        compute(bufs_b)
        return carry

    lax.fori_loop(0, NSC // 2, pair_loop, 0)
    drain(bufs_a, sem_a)

    pltpu.sync_copy(sacc, agg_hbm.at[pl.ds(wid * N, N)])
    pltpu.sync_copy(macc, agg_hbm.at[pl.ds((32 + wid) * N, N)])


def _out_mlp_body(nf_ref, agg0_ref, agg1_ref,
                  wa_ref, wb_ref, wc_ref, b1_ref,
                  w2_ref, b2_ref, w3_ref, b3_ref, w4_ref, b4_ref,
                  out_ref):
    h1 = jnp.dot(nf_ref[...], wa_ref[...], preferred_element_type=jnp.float32)
    sumt = agg0_ref[:32, :] + agg1_ref[:32, :]
    h1 += lax.dot_general(sumt, wb_ref[...], (((0,), (0,)), ((), ())),
                          preferred_element_type=jnp.float32)
    mx = jnp.maximum(agg0_ref[32:, :], agg1_ref[32:, :])
    mx = jnp.where(jnp.isinf(mx) & (mx < 0), 0.0, mx)
    h1 += lax.dot_general(mx, wc_ref[...], (((0,), (0,)), ((), ())),
                          preferred_element_type=jnp.float32)
    h1 = _leaky(h1 + b1_ref[...])
    h2 = _leaky(jnp.dot(h1, w2_ref[...], preferred_element_type=jnp.float32)
                + b2_ref[...])
    h3 = _leaky(jnp.dot(h2, w3_ref[...], preferred_element_type=jnp.float32)
                + b3_ref[...])
    out_ref[...] = jnp.dot(h3, w4_ref[...],
                           preferred_element_type=jnp.float32) + b4_ref[...]


def _out_mlp_call(nf, agg0, agg1, wa, wb, wc, b1, w2, b2, w3, b3, w4, b4):
    return pl.pallas_call(
        _out_mlp_body,
        out_shape=jax.ShapeDtypeStruct((N, OUT_NF), jnp.float32),
    )(nf, agg0, agg1, wa, wb, wc, b1, w2, b2, w3, b3, w4, b4)


@jax.jit
def kernel(nf, ef, msg_params, red_params, edge_index):
    w1, b1, w2, b2, w3, b3, w4, b4 = msg_params
    wr1, br1, wr2, br2, wr3, br3, wr4, br4 = red_params

    src = edge_index[0].astype(jnp.int32)
    dst = edge_index[1].astype(jnp.int32)

    w1s, w1d, w1e = w1[:IN_NF], w1[IN_NF:2 * IN_NF], w1[2 * IN_NF:]
    w4k, w4f = w4[:, :1], w4[:, 1:]
    b4k, b4f = b4[:1].reshape(1, 1), b4[1:].reshape(64, 1)

    p, q = _pq_call(nf, w1s, w1d)
    aggs = []
    for s in range(NSLAB):
        src_s = src[s * SE:(s + 1) * SE]
        dst_s = dst[s * SE:(s + 1) * SE]
        ps, qd = _gather_kernel_fn()(p, q, src_s, dst_s)
        eft = _edge_mlp_call(ps, qd, ef[s * SE:(s + 1) * SE],
                             w1e, b1.reshape(1, 64), w2, b2.reshape(1, 128),
                             w3, b3.reshape(1, 64), w4k, b4k, w4f, b4f)
        aggs.append(_scatter_kernel_fn()(eft.reshape(64 * SE), dst_s))

    wa, wb, wc = wr1[:IN_NF], wr1[IN_NF:IN_NF + 32], wr1[IN_NF + 32:]
    return _out_mlp_call(nf, aggs[0].reshape(64, N), aggs[1].reshape(64, N),
                         wa, wb, wc, br1.reshape(1, 64),
                         wr2, br2.reshape(1, 128), wr3, br3.reshape(1, 64),
                         wr4, br4.reshape(1, OUT_NF))


# speedup vs baseline: 3.2614x; 1.0238x over previous
import functools

import jax
import jax.numpy as jnp
from jax import lax
from jax.experimental import pallas as pl
from jax.experimental.pallas import tpu as pltpu
from jax.experimental.pallas import tpu_sc as plsc

N = 10000
E = 320000
IN_NF = 128
IN_EF = 16
H1 = 32
H2 = 32
OUT_NF = 128

NC, NS, LANES = 2, 16, 16
NW = NC * NS

NSLAB = 2
SE = E // NSLAB
EPW = SE // NW
GC = 128
NGC = EPW // GC
GTAIL = EPW - NGC * GC

SC_CHUNK = 2000
NSC = SE // SC_CHUNK
NGRP = SC_CHUNK // LANES

BE = 1280

def _mesh():
    return plsc.VectorSubcoreMesh(
        core_axis_name="c", subcore_axis_name="s",
        num_cores=NC, num_subcores=NS)


def _leaky(x):
    return jnp.where(x >= 0, x, 0.2 * x)


def _pq_body(nf_ref, ws_ref, wd_ref, p_ref, q_ref):
    nf = nf_ref[...]
    p_ref[...] = jnp.dot(nf, ws_ref[...], preferred_element_type=jnp.float32)
    q_ref[...] = jnp.dot(nf, wd_ref[...], preferred_element_type=jnp.float32)


def _pq_call(nf, w1s, w1d):
    return pl.pallas_call(
        _pq_body,
        out_shape=[jax.ShapeDtypeStruct((N, 64), jnp.float32)] * 2,
    )(nf, w1s, w1d)


@functools.cache
def _gather_kernel_fn():
    return pl.kernel(
        _gather_body,
        out_type=[jax.ShapeDtypeStruct((SE, 64), jnp.float32)] * 2,
        mesh=_mesh(),
        scratch_types=[
            pltpu.VMEM((GC,), jnp.int32),
            pltpu.VMEM((GC,), jnp.int32),
            pltpu.VMEM((GC, 64), jnp.float32),
            pltpu.VMEM((GC, 64), jnp.float32),
            pltpu.VMEM((GC,), jnp.int32),
            pltpu.VMEM((GC,), jnp.int32),
            pltpu.VMEM((GC, 64), jnp.float32),
            pltpu.VMEM((GC, 64), jnp.float32),
            pltpu.SemaphoreType.DMA,
            pltpu.SemaphoreType.DMA,
            pltpu.SemaphoreType.DMA,
            pltpu.SemaphoreType.DMA,
        ],
        compiler_params=pltpu.CompilerParams(use_tc_tiling_on_sc=False),
    )


def _gather_body(p_hbm, q_hbm, src_hbm, dst_hbm, ps_hbm, qd_hbm,
                 sidx_a, didx_a, prow_a, qrow_a,
                 sidx_b, didx_b, prow_b, qrow_b,
                 gsem_a, gsem_b, wsem_a, wsem_b):
    wid = lax.axis_index("s") * NC + lax.axis_index("c")
    base0 = wid * EPW
    A = (sidx_a, didx_a, prow_a, qrow_a, gsem_a, wsem_a)
    B = (sidx_b, didx_b, prow_b, qrow_b, gsem_b, wsem_b)

    def load_and_gather(base, bufs):
        sidx, didx, prow, qrow, gsem, _ = bufs
        pltpu.sync_copy(src_hbm.at[pl.ds(base, GC)], sidx)
        pltpu.sync_copy(dst_hbm.at[pl.ds(base, GC)], didx)
        pltpu.async_copy(p_hbm.at[sidx], prow, gsem)
        pltpu.async_copy(q_hbm.at[didx], qrow, gsem)

    def finish_and_write(base, bufs):
        sidx, didx, prow, qrow, gsem, wsem = bufs
        pltpu.make_async_copy(p_hbm.at[sidx], prow, gsem).wait()
        pltpu.make_async_copy(q_hbm.at[didx], qrow, gsem).wait()
        pltpu.async_copy(prow, ps_hbm.at[pl.ds(base, GC)], wsem)
        pltpu.async_copy(qrow, qd_hbm.at[pl.ds(base, GC)], wsem)

    def drain_writes(bufs):
        _, _, prow, qrow, _, wsem = bufs
        pltpu.make_async_copy(prow, ps_hbm.at[pl.ds(0, GC)], wsem).wait()
        pltpu.make_async_copy(qrow, qd_hbm.at[pl.ds(0, GC)], wsem).wait()

    def pair(ci, carry):
        c0 = base0 + (2 * ci) * GC
        c1 = c0 + GC

        @pl.when(ci > 0)
        def _():
            drain_writes(A)
            drain_writes(B)

        load_and_gather(c0, A)
        load_and_gather(c1, B)
        finish_and_write(c0, A)
        finish_and_write(c1, B)
        return carry

    lax.fori_loop(0, NGC // 2, pair, 0)
    drain_writes(A)
    drain_writes(B)

    def do_chunk(base, n):
        sidx, didx, prow, qrow, gsem, _ = A
        pltpu.sync_copy(src_hbm.at[pl.ds(base, n)], sidx.at[pl.ds(0, n)])
        pltpu.sync_copy(dst_hbm.at[pl.ds(base, n)], didx.at[pl.ds(0, n)])
        cp1 = pltpu.async_copy(p_hbm.at[sidx.at[pl.ds(0, n)]],
                               prow.at[pl.ds(0, n)], gsem)
        cp2 = pltpu.async_copy(q_hbm.at[didx.at[pl.ds(0, n)]],
                               qrow.at[pl.ds(0, n)], gsem)
        cp1.wait()
        cp2.wait()
        pltpu.sync_copy(prow.at[pl.ds(0, n)], ps_hbm.at[pl.ds(base, n)])
        pltpu.sync_copy(qrow.at[pl.ds(0, n)], qd_hbm.at[pl.ds(base, n)])

    if NGC % 2:
        do_chunk(base0 + (NGC - 1) * GC, GC)
    if GTAIL:
        do_chunk(base0 + NGC * GC, GTAIL)


def _edge_mlp_body(ps_ref, qd_ref, ef_ref,
                   w1e_ref, b1_ref, w2_ref, b2_ref, w3_ref, b3_ref,
                   w4k_ref, b4k_ref, w4f_ref, b4f_ref,
                   out_ref):
    h1 = ps_ref[...] + qd_ref[...]
    h1 += jnp.dot(ef_ref[...], w1e_ref[...], preferred_element_type=jnp.float32)
    h1 = _leaky(h1 + b1_ref[...])
    h2 = _leaky(jnp.dot(h1, w2_ref[...], preferred_element_type=jnp.float32)
                + b2_ref[...])
    h3 = _leaky(jnp.dot(h2, w3_ref[...], preferred_element_type=jnp.float32)
                + b3_ref[...])
    kT = lax.dot_general(w4k_ref[...], h3, (((0,), (1,)), ((), ())),
                         preferred_element_type=jnp.float32)
    kT = 1.0 / (1.0 + jnp.exp(-(kT + b4k_ref[...])))
    fT = lax.dot_general(w4f_ref[...], h3, (((0,), (1,)), ((), ())),
                         preferred_element_type=jnp.float32)
    out_ref[...] = (fT + b4f_ref[...]) * kT


def _edge_mlp_call(ps, qd, ef, w1e, b1, w2, b2, w3, b3, w4k, b4k, w4f, b4f):
    grid = SE // BE
    wspec = lambda shape: pl.BlockSpec(shape, lambda i: (0,) * len(shape))
    return pl.pallas_call(
        _edge_mlp_body,
        grid=(grid,),
        in_specs=[
            pl.BlockSpec((BE, 64), lambda i: (i, 0)),
            pl.BlockSpec((BE, 64), lambda i: (i, 0)),
            pl.BlockSpec((BE, IN_EF), lambda i: (i, 0)),
            wspec((IN_EF, 64)), wspec((1, 64)),
            wspec((64, 128)), wspec((1, 128)),
            wspec((128, 64)), wspec((1, 64)),
            wspec((64, 1)), wspec((1, 1)),
            wspec((64, 64)), wspec((64, 1)),
        ],
        out_specs=pl.BlockSpec((64, BE), lambda i: (0, i)),
        out_shape=jax.ShapeDtypeStruct((64, SE), jnp.float32),
    )(ps, qd, ef, w1e, b1, w2, b2, w3, b3, w4k, b4k, w4f, b4f)


@functools.cache
def _scatter_kernel_fn():
    return pl.kernel(
        _scatter_body,
        out_type=jax.ShapeDtypeStruct((64 * N,), jnp.float32),
        mesh=_mesh(),
        scratch_types=[
            pltpu.VMEM((N,), jnp.float32),
            pltpu.VMEM((N,), jnp.float32),
            pltpu.VMEM((SC_CHUNK,), jnp.int32),
            pltpu.VMEM((SC_CHUNK,), jnp.float32),
            pltpu.VMEM((SC_CHUNK,), jnp.float32),
            pltpu.VMEM((SC_CHUNK,), jnp.int32),
            pltpu.VMEM((SC_CHUNK,), jnp.float32),
            pltpu.VMEM((SC_CHUNK,), jnp.float32),
            pltpu.SemaphoreType.DMA,
            pltpu.SemaphoreType.DMA,
        ],
        compiler_params=pltpu.CompilerParams(needs_layout_passes=False),
    )


def _scatter_body(eft_hbm, dst_hbm, agg_hbm, sacc, macc,
                  didx_a, v1_a, v2_a, didx_b, v1_b, v2_b, sem_a, sem_b):
    wid = lax.axis_index("s") * NC + lax.axis_index("c")

    def init_loop(i, carry):
        sacc[pl.ds(i * LANES, LANES)] = jnp.zeros((LANES,), jnp.float32)
        macc[pl.ds(i * LANES, LANES)] = jnp.full((LANES,), -jnp.inf,
                                                 jnp.float32)
        return carry

    lax.fori_loop(0, N // LANES, init_loop, 0)

    sum_off = wid * SE
    max_off = (32 + wid) * SE

    def issue(base, bufs, sem):
        bd, b1, b2 = bufs
        pltpu.async_copy(dst_hbm.at[pl.ds(base, SC_CHUNK)], bd, sem)
        pltpu.async_copy(eft_hbm.at[pl.ds(sum_off + base, SC_CHUNK)], b1, sem)
        pltpu.async_copy(eft_hbm.at[pl.ds(max_off + base, SC_CHUNK)], b2, sem)

    def drain(bufs, sem):
        bd, b1, b2 = bufs
        pltpu.make_async_copy(dst_hbm.at[pl.ds(0, SC_CHUNK)], bd, sem).wait()
        pltpu.make_async_copy(eft_hbm.at[pl.ds(0, SC_CHUNK)], b1, sem).wait()
        pltpu.make_async_copy(eft_hbm.at[pl.ds(0, SC_CHUNK)], b2, sem).wait()

    def compute(bufs):
        bd, b1, b2 = bufs
        def one_group(off, fail):
            sl = pl.ds(off, LANES)
            d = bd[sl]
            s = b1[sl]
            m = b2[sl]
            plsc.addupdate_scatter(sacc, [d], s)
            cur = plsc.load_gather(macc, [d])
            new = jnp.maximum(cur, m)
            plsc.store_scatter(macc, [d], new)
            cur2 = plsc.load_gather(macc, [d])
            plsc.store_scatter(macc, [d], jnp.maximum(cur2, new),
                              mask=cur2 < new)
            chk = plsc.load_gather(macc, [d])
            return fail | (chk < new)

        def grp(gi, fail):
            fail = one_group(gi * (2 * LANES), fail)
            return one_group(gi * (2 * LANES) + LANES, fail)

        fail = lax.fori_loop(0, NGRP // 2, grp,
                             jnp.zeros((LANES,), jnp.bool_))
        if NGRP % 2:
            fail = one_group((NGRP - 1) * LANES, fail)

        @pl.when(jnp.any(fail))
        def _fixup():
            def grp2(gi, carry):
                sl = pl.ds(gi * LANES, LANES)
                d = bd[sl]
                m = b2[sl]

                def retry_body(need):
                    c2 = plsc.load_gather(macc, [d])
                    plsc.store_scatter(macc, [d], jnp.maximum(c2, m),
                                      mask=need)
                    c3 = plsc.load_gather(macc, [d])
                    return c3 < m

                lax.while_loop(lambda n: jnp.any(n), retry_body,
                               plsc.load_gather(macc, [d]) < m)
                return carry

            lax.fori_loop(0, NGRP, grp2, 0)

    bufs_a = (didx_a, v1_a, v2_a)
    bufs_b = (didx_b, v1_b, v2_b)
    issue(0, bufs_a, sem_a)

    def pair_loop(ci, carry):
        base_a = (2 * ci) * SC_CHUNK
        base_b = base_a + SC_CHUNK
        base_n = jnp.minimum(base_a + 2 * SC_CHUNK, SE - SC_CHUNK)
        drain(bufs_a, sem_a)
        issue(base_b, bufs_b, sem_b)
        compute(bufs_a)
        drain(bufs_b, sem_b)
        issue(base_n, bufs_a, sem_a)
        compute(bufs_b)
        return carry

    lax.fori_loop(0, NSC // 2, pair_loop, 0)
    drain(bufs_a, sem_a)

    pltpu.sync_copy(sacc, agg_hbm.at[pl.ds(wid * N, N)])
    pltpu.sync_copy(macc, agg_hbm.at[pl.ds((32 + wid) * N, N)])


def _out_mlp_body(nf_ref, agg0_ref, agg1_ref,
                  wa_ref, wb_ref, wc_ref, b1_ref,
                  w2_ref, b2_ref, w3_ref, b3_ref, w4_ref, b4_ref,
                  out_ref):
    h1 = jnp.dot(nf_ref[...], wa_ref[...], preferred_element_type=jnp.float32)
    sumt = agg0_ref[:32, :] + agg1_ref[:32, :]
    h1 += lax.dot_general(sumt, wb_ref[...], (((0,), (0,)), ((), ())),
                          preferred_element_type=jnp.float32)
    mx = jnp.maximum(agg0_ref[32:, :], agg1_ref[32:, :])
    mx = jnp.where(jnp.isinf(mx) & (mx < 0), 0.0, mx)
    h1 += lax.dot_general(mx, wc_ref[...], (((0,), (0,)), ((), ())),
                          preferred_element_type=jnp.float32)
    h1 = _leaky(h1 + b1_ref[...])
    h2 = _leaky(jnp.dot(h1, w2_ref[...], preferred_element_type=jnp.float32)
                + b2_ref[...])
    h3 = _leaky(jnp.dot(h2, w3_ref[...], preferred_element_type=jnp.float32)
                + b3_ref[...])
    out_ref[...] = jnp.dot(h3, w4_ref[...],
                           preferred_element_type=jnp.float32) + b4_ref[...]


def _out_mlp_call(nf, agg0, agg1, wa, wb, wc, b1, w2, b2, w3, b3, w4, b4):
    return pl.pallas_call(
        _out_mlp_body,
        out_shape=jax.ShapeDtypeStruct((N, OUT_NF), jnp.float32),
    )(nf, agg0, agg1, wa, wb, wc, b1, w2, b2, w3, b3, w4, b4)


@jax.jit
def kernel(nf, ef, msg_params, red_params, edge_index):
    w1, b1, w2, b2, w3, b3, w4, b4 = msg_params
    wr1, br1, wr2, br2, wr3, br3, wr4, br4 = red_params

    src = edge_index[0].astype(jnp.int32)
    dst = edge_index[1].astype(jnp.int32)

    w1s, w1d, w1e = w1[:IN_NF], w1[IN_NF:2 * IN_NF], w1[2 * IN_NF:]
    w4k, w4f = w4[:, :1], w4[:, 1:]
    b4k, b4f = b4[:1].reshape(1, 1), b4[1:].reshape(64, 1)

    p, q = _pq_call(nf, w1s, w1d)
    aggs = []
    for s in range(NSLAB):
        src_s = src[s * SE:(s + 1) * SE]
        dst_s = dst[s * SE:(s + 1) * SE]
        ps, qd = _gather_kernel_fn()(p, q, src_s, dst_s)
        eft = _edge_mlp_call(ps, qd, ef[s * SE:(s + 1) * SE],
                             w1e, b1.reshape(1, 64), w2, b2.reshape(1, 128),
                             w3, b3.reshape(1, 64), w4k, b4k, w4f, b4f)
        aggs.append(_scatter_kernel_fn()(eft.reshape(64 * SE), dst_s))

    wa, wb, wc = wr1[:IN_NF], wr1[IN_NF:IN_NF + 32], wr1[IN_NF + 32:]
    return _out_mlp_call(nf, aggs[0].reshape(64, N), aggs[1].reshape(64, N),
                         wa, wb, wc, br1.reshape(1, 64),
                         wr2, br2.reshape(1, 128), wr3, br3.reshape(1, 64),
                         wr4, br4.reshape(1, OUT_NF))
